# baseline (speedup 1.0000x reference)
import functools
import math

import jax
import jax.numpy as jnp
from jax import lax
from jax.experimental import pallas as pl
from jax.experimental.pallas import tpu as pltpu
from jax.experimental.pallas import tpu_sc as plsc

_SEQ = 4096
_BATCH = 4
_DIM = 64
_VOCAB = 1000000
_SCALE = math.sqrt(_DIM)

_NW = 32
_STRIPE = 31232
_CHW = 256
_NCH = _STRIPE // _CHW
_ROWS = _SEQ * _BATCH
_TRASH = _ROWS
_OUT1R = _ROWS + 8

_MESH = plsc.VectorSubcoreMesh(core_axis_name="c", subcore_axis_name="s")
_PARAMS = pltpu.CompilerParams(use_tc_tiling_on_sc=True, needs_layout_passes=False)


def _iota16():
    return lax.iota(jnp.int32, 16)


def _count(m):
    return plsc.all_reduce_population_count(m)[0]


@functools.partial(
    pl.kernel,
    out_type=jax.ShapeDtypeStruct((_OUT1R, 128), jnp.float32),
    mesh=_MESH,
    scratch_types=[
        pltpu.VMEM((_BATCH, _SEQ), jnp.int32),
        pltpu.VMEM((_ROWS,), jnp.int32),
        pltpu.VMEM((_ROWS,), jnp.int32),
        pltpu.VMEM((2, _DIM, _CHW), jnp.float32),
        pltpu.VMEM((_DIM, 64), jnp.float32),
        pltpu.VMEM((128, 16), jnp.float32),
        pltpu.VMEM((16, 128), jnp.float32),
        pltpu.VMEM((16,), jnp.int32),
        pltpu.SemaphoreType.DMA((2,)),
        pltpu.SemaphoreType.DMA,
    ],
    compiler_params=_PARAMS,
)
def _gather_pass(x_hbm, tab_hbm, tail_hbm, out_hbm, idxv, fbuf, cbuf, chv,
                 tailv, abuf, rowb, flist, sem, sem2):
    wid = lax.axis_index("s") * 2 + lax.axis_index("c")
    wlo = wid * _STRIPE
    whi = jnp.where(wid == _NW - 1, _VOCAB, wlo + _STRIPE)
    i16 = _iota16()

    pltpu.sync_copy(x_hbm, idxv)

    def prefilter_b(b, n):
        def scan_g(g, n):
            v0 = idxv[b, pl.ds(g * 32, 16)]
            v1 = idxv[b, pl.ds(g * 32 + 16, 16)]
            f0 = (g * 32 + i16) * _BATCH + b
            f1 = (g * 32 + 16 + i16) * _BATCH + b
            m0 = (v0 >= wlo) & (v0 < whi)
            m1 = (v1 >= wlo) & (v1 < whi)
            c0 = _count(m0)
            c1 = _count(m1)
            plsc.store_compressed(fbuf.at[pl.ds(n, 16)], f0, mask=m0)
            plsc.store_compressed(fbuf.at[pl.ds(n + c0, 16)], f1, mask=m1)
            return n + c0 + c1
        return lax.fori_loop(0, _SEQ // 32, scan_g, n)

    n = lax.fori_loop(0, _BATCH, prefilter_b, 0)

    def scan_extract(src_ref, cs, hi):
        def scan_q(q, cnt):
            fr0 = fbuf[pl.ds(q * 32, 16)]
            fr1 = fbuf[pl.ds(q * 32 + 16, 16)]
            valid0 = (q * 32 + i16) < n
            valid1 = (q * 32 + 16 + i16) < n
            fq0 = jnp.where(valid0, fr0, 0)
            fq1 = jnp.where(valid1, fr1, 0)
            v0 = plsc.load_gather(idxv, [fq0 & 3, fq0 >> 2], mask=valid0)
            v1 = plsc.load_gather(idxv, [fq1 & 3, fq1 >> 2], mask=valid1)
            m0 = valid0 & (v0 >= cs) & (v0 < hi)
            m1 = valid1 & (v1 >= cs) & (v1 < hi)
            c0 = _count(m0)
            c1 = _count(m1)
            plsc.store_compressed(cbuf.at[pl.ds(cnt, 16)], fq0, mask=m0)
            plsc.store_compressed(cbuf.at[pl.ds(cnt + c0, 16)], fq1, mask=m1)
            return cnt + c0 + c1

        cnt = lax.fori_loop(0, (n + 31) // 32, scan_q, 0)

        def ext(e, carry):
            fq_r = cbuf[pl.ds(e * 16, 16)]
            valid = (e * 16 + i16) < cnt
            fq = jnp.where(valid, fq_r, 0)
            v = plsc.load_gather(idxv, [fq & 3, fq >> 2], mask=valid)
            lv = jnp.where(valid, v - cs, 0)
            for d in range(_DIM):
                g = plsc.load_gather(src_ref, [jnp.full((16,), d, jnp.int32), lv],
                                     mask=valid)
                abuf[d, :] = g
            flist[...] = jnp.where(valid, fq_r, _TRASH)
            for j in range(16):
                for cg in range(_DIM // 16):
                    rowb[j, pl.ds(cg * 16, 16)] = plsc.load_gather(
                        abuf, [cg * 16 + i16, jnp.full((16,), j, jnp.int32)])
            pltpu.async_copy(rowb, out_hbm.at[flist], sem2).wait()
            return carry

        lax.fori_loop(0, (cnt + 15) // 16, ext, 0)

    nch = _NCH + jnp.where(wid == _NW - 1, 1, 0)

    pltpu.async_copy(tab_hbm.at[:, pl.ds(wlo, _CHW)], chv.at[0], sem.at[0])

    def chunk_body(c, carry):
        p = lax.rem(c, 2)

        @pl.when(c + 1 < nch)
        def _():
            csn = pl.multiple_of(wlo + (c + 1) * _CHW, 128)
            pltpu.async_copy(tab_hbm.at[:, pl.ds(csn, _CHW)], chv.at[1 - p],
                             sem.at[1 - p])

        cs = pl.multiple_of(wlo + c * _CHW, 128)
        pltpu.make_async_copy(tab_hbm.at[:, pl.ds(cs, _CHW)], chv.at[p],
                              sem.at[p]).wait()
        scan_extract(chv.at[p], cs, cs + _CHW)
        return carry

    lax.fori_loop(0, nch, chunk_body, 0)

    @pl.when(wid == _NW - 1)
    def _():
        pltpu.sync_copy(tail_hbm, tailv)
        scan_extract(tailv, _VOCAB - 64, _VOCAB)


@functools.partial(
    pl.kernel,
    out_type=jax.ShapeDtypeStruct((_BATCH, _DIM, _SEQ), jnp.float32),
    mesh=_MESH,
    scratch_types=[
        pltpu.VMEM((512, 128), jnp.float32),
        pltpu.VMEM((_DIM, 128), jnp.float32),
        pltpu.VMEM((_BATCH, _DIM, 128), jnp.float32),
    ],
    compiler_params=_PARAMS,
)
def _finish_pass(rows_hbm, pe_hbm, out_hbm, rv, pv, ov):
    wid = lax.axis_index("s") * 2 + lax.axis_index("c")
    s0 = wid * 128
    i16 = _iota16()

    pltpu.sync_copy(rows_hbm.at[pl.ds(wid * 512, 512)], rv)
    pltpu.sync_copy(pe_hbm.at[:, pl.ds(s0, 128)], pv)

    def body(d, carry):
        dsplat = jnp.full((16,), d, jnp.int32)
        for b in range(_BATCH):
            for sg in range(128 // 16):
                fl = (sg * 16 + i16) * _BATCH + b
                raw = plsc.load_gather(rv, [fl, dsplat])
                ov[b, d, pl.ds(sg * 16, 16)] = raw * _SCALE + pv[d, pl.ds(sg * 16, 16)]
        return carry

    lax.fori_loop(0, _DIM, body, 0)

    pltpu.sync_copy(ov, out_hbm.at[:, :, pl.ds(s0, 128)])


def kernel(x, table, pe):
    tab_t = table.T
    raw = _gather_pass(x.T, tab_t, tab_t[:, _VOCAB - 64:])
    out_t = _finish_pass(raw, pe[:, 0, :].T)
    return jnp.transpose(out_t, (2, 0, 1))


# speedup vs baseline: 2.2275x; 2.2275x over previous
import functools
import math

import jax
import jax.numpy as jnp
from jax import lax
from jax.experimental import pallas as pl
from jax.experimental.pallas import tpu as pltpu
from jax.experimental.pallas import tpu_sc as plsc

_SEQ = 4096
_BATCH = 4
_DIM = 64
_VOCAB = 1000000
_SCALE = math.sqrt(_DIM)

_NW = 32
_STRIPE = 31232
_CHW = 512
_NCH = _STRIPE // _CHW
_ROWS = _SEQ * _BATCH
_TRASH = _ROWS
_OUT1R = _ROWS + 8

_MESH = plsc.VectorSubcoreMesh(core_axis_name="c", subcore_axis_name="s")
_PARAMS = pltpu.CompilerParams(use_tc_tiling_on_sc=True, needs_layout_passes=False)


def _iota16():
    return lax.iota(jnp.int32, 16)


def _count(m):
    return jnp.sum(jnp.where(m, 1, 0))


@functools.partial(
    pl.kernel,
    out_type=jax.ShapeDtypeStruct((_OUT1R, 128), jnp.float32),
    mesh=_MESH,
    scratch_types=[
        pltpu.VMEM((_BATCH, _SEQ), jnp.int32),
        pltpu.VMEM((_ROWS,), jnp.int32),
        pltpu.VMEM((_ROWS,), jnp.int32),
        pltpu.VMEM((_DIM, _CHW), jnp.float32),
        pltpu.VMEM((_DIM, 64), jnp.float32),
        pltpu.VMEM((128, 16), jnp.float32),
        pltpu.VMEM((16, 128), jnp.float32),
        pltpu.VMEM((16,), jnp.int32),
        pltpu.SemaphoreType.DMA,
    ],
    compiler_params=_PARAMS,
)
def _gather_pass(x_hbm, tab_hbm, tail_hbm, out_hbm, idxv, fbuf, cbuf, chv,
                 tailv, abuf, rowb, flist, sem2):
    wid = lax.axis_index("s") * 2 + lax.axis_index("c")
    wlo = wid * _STRIPE
    whi = jnp.where(wid == _NW - 1, _VOCAB, wlo + _STRIPE)
    i16 = _iota16()

    pltpu.sync_copy(x_hbm, idxv)

    def prefilter_b(b, n):
        def scan_g(g, n):
            v0 = idxv[b, pl.ds(g * 32, 16)]
            v1 = idxv[b, pl.ds(g * 32 + 16, 16)]
            f0 = (g * 32 + i16) * _BATCH + b
            f1 = (g * 32 + 16 + i16) * _BATCH + b
            m0 = (v0 >= wlo) & (v0 < whi)
            m1 = (v1 >= wlo) & (v1 < whi)
            c0 = _count(m0)
            c1 = _count(m1)
            plsc.store_compressed(fbuf.at[pl.ds(n, 16)], f0, mask=m0)
            plsc.store_compressed(fbuf.at[pl.ds(n + c0, 16)], f1, mask=m1)
            return n + c0 + c1
        return lax.fori_loop(0, _SEQ // 32, scan_g, n)

    n = lax.fori_loop(0, _BATCH, prefilter_b, 0)

    def scan_extract(src_ref, cs, hi):
        def scan_q(q, cnt):
            fr0 = fbuf[pl.ds(q * 32, 16)]
            fr1 = fbuf[pl.ds(q * 32 + 16, 16)]
            valid0 = (q * 32 + i16) < n
            valid1 = (q * 32 + 16 + i16) < n
            fq0 = jnp.where(valid0, fr0, 0)
            fq1 = jnp.where(valid1, fr1, 0)
            v0 = plsc.load_gather(idxv, [fq0 & 3, fq0 >> 2], mask=valid0)
            v1 = plsc.load_gather(idxv, [fq1 & 3, fq1 >> 2], mask=valid1)
            m0 = valid0 & (v0 >= cs) & (v0 < hi)
            m1 = valid1 & (v1 >= cs) & (v1 < hi)
            c0 = _count(m0)
            c1 = _count(m1)
            plsc.store_compressed(cbuf.at[pl.ds(cnt, 16)], fq0, mask=m0)
            plsc.store_compressed(cbuf.at[pl.ds(cnt + c0, 16)], fq1, mask=m1)
            return cnt + c0 + c1

        cnt = lax.fori_loop(0, (n + 31) // 32, scan_q, 0)

        def ext(e, carry):
            fq_r = cbuf[pl.ds(e * 16, 16)]
            valid = (e * 16 + i16) < cnt
            fq = jnp.where(valid, fq_r, 0)
            v = plsc.load_gather(idxv, [fq & 3, fq >> 2], mask=valid)
            lv = jnp.where(valid, v - cs, 0)
            for d in range(_DIM):
                g = plsc.load_gather(src_ref, [jnp.full((16,), d, jnp.int32), lv],
                                     mask=valid)
                abuf[d, :] = g
            flist[...] = jnp.where(valid, fq_r, _TRASH)
            for j in range(16):
                for cg in range(_DIM // 16):
                    rowb[j, pl.ds(cg * 16, 16)] = plsc.load_gather(
                        abuf, [cg * 16 + i16, jnp.full((16,), j, jnp.int32)])
            pltpu.async_copy(rowb, out_hbm.at[flist], sem2).wait()
            return carry

        lax.fori_loop(0, (cnt + 15) // 16, ext, 0)

    nch = _NCH + jnp.where(wid == _NW - 1, 1, 0)

    def chunk_body(c, carry):
        cs = pl.multiple_of(wlo + c * _CHW, 128)
        pltpu.sync_copy(tab_hbm.at[:, pl.ds(cs, _CHW)], chv)
        scan_extract(chv, cs, cs + _CHW)
        return carry

    lax.fori_loop(0, nch, chunk_body, 0)

    @pl.when(wid == _NW - 1)
    def _():
        pltpu.sync_copy(tail_hbm, tailv)
        scan_extract(tailv, _VOCAB - 64, _VOCAB)


@functools.partial(
    pl.kernel,
    out_type=jax.ShapeDtypeStruct((_BATCH, _DIM, _SEQ), jnp.float32),
    mesh=_MESH,
    scratch_types=[
        pltpu.VMEM((512, 128), jnp.float32),
        pltpu.VMEM((_DIM, 128), jnp.float32),
        pltpu.VMEM((_BATCH, _DIM, 128), jnp.float32),
    ],
    compiler_params=_PARAMS,
)
def _finish_pass(rows_hbm, pe_hbm, out_hbm, rv, pv, ov):
    wid = lax.axis_index("s") * 2 + lax.axis_index("c")
    s0 = wid * 128
    i16 = _iota16()

    pltpu.sync_copy(rows_hbm.at[pl.ds(wid * 512, 512)], rv)
    pltpu.sync_copy(pe_hbm.at[:, pl.ds(s0, 128)], pv)

    def body(d, carry):
        dsplat = jnp.full((16,), d, jnp.int32)
        for b in range(_BATCH):
            for sg in range(128 // 16):
                fl = (sg * 16 + i16) * _BATCH + b
                raw = plsc.load_gather(rv, [fl, dsplat])
                ov[b, d, pl.ds(sg * 16, 16)] = raw * _SCALE + pv[d, pl.ds(sg * 16, 16)]
        return carry

    lax.fori_loop(0, _DIM, body, 0)

    pltpu.sync_copy(ov, out_hbm.at[:, :, pl.ds(s0, 128)])


def kernel(x, table, pe):
    tab_t = table.T
    raw = _gather_pass(x.T, tab_t, tab_t[:, _VOCAB - 64:])
    out_t = _finish_pass(raw, pe[:, 0, :].T)
    return jnp.transpose(out_t, (2, 0, 1))
